# Initial kernel scaffold; baseline (speedup 1.0000x reference)
import jax
import jax.numpy as jnp
from jax.experimental import pallas as pl


def kernel(x, edge, W1, b1, W2, b2, W3, b3):
    raise NotImplementedError("write your pallas kernel here")


# speedup vs baseline: 6.1039x; 6.1039x over previous
import functools

import jax
import jax.numpy as jnp
from jax import lax
from jax.experimental import pallas as pl
from jax.experimental.pallas import tpu as pltpu
from jax.experimental.pallas import tpu_sc as plsc

_N = 10000
_E = 160000
_D = 256
_K = 5000
_TAU = 0.1
_NP = 10240
_EPAD = 163840
_BLK = 1000


def _mm_split_body(relu_in, x_ref, w_ref, b_ref, o0, o1, o2, o3):
    xb = x_ref[...]
    if relu_in:
        xb = jnp.maximum(xb, 0.0)
    g = jnp.dot(xb, w_ref[...], preferred_element_type=jnp.float32) + b_ref[...]
    o0[...] = g[:, 0:64]
    o1[...] = g[:, 64:128]
    o2[...] = g[:, 128:192]
    o3[...] = g[:, 192:256]


def _mm_split(x, w, b, relu_in):
    return pl.pallas_call(
        functools.partial(_mm_split_body, relu_in),
        grid=(_N // _BLK,),
        in_specs=[
            pl.BlockSpec((_BLK, _D), lambda g: (g, 0)),
            pl.BlockSpec((_D, _D), lambda g: (0, 0)),
            pl.BlockSpec((1, _D), lambda g: (0, 0)),
        ],
        out_specs=[pl.BlockSpec((_BLK, 64), lambda g: (g, 0))] * 4,
        out_shape=[jax.ShapeDtypeStruct((_N, 64), jnp.float32)] * 4,
    )(x, w, b.reshape(1, _D))


def _mm3_body(a_ref, w_ref, b_ref, h_ref, s_ref):
    h = jnp.maximum(a_ref[...], 0.0)
    h_ref[...] = h
    s_ref[...] = jnp.dot(h, w_ref[...], preferred_element_type=jnp.float32) + b_ref[...]


def _mm3(a, w3, b3):
    return pl.pallas_call(
        _mm3_body,
        grid=(_N // _BLK,),
        in_specs=[
            pl.BlockSpec((_BLK, _D), lambda g: (g, 0)),
            pl.BlockSpec((_D, 1), lambda g: (0, 0)),
            pl.BlockSpec((1, 1), lambda g: (0, 0)),
        ],
        out_specs=[
            pl.BlockSpec((_BLK, _D), lambda g: (g, 0)),
            pl.BlockSpec((_BLK, 1), lambda g: (g, 0)),
        ],
        out_shape=[
            jax.ShapeDtypeStruct((_N, _D), jnp.float32),
            jax.ShapeDtypeStruct((_N, 1), jnp.float32),
        ],
    )(a, w3, b3.reshape(1, 1))


def _topk_body(p_ref, ns_ref, mask_ref):
    p = p_ref[0] + p_ref[1]
    ns_ref[...] = p
    bits = lax.bitcast_convert_type(p, jnp.uint32)
    neg = (bits >> jnp.uint32(31)) == jnp.uint32(1)
    key = jnp.where(neg, ~bits, bits | jnp.uint32(0x80000000))
    row = lax.broadcasted_iota(jnp.int32, (80, 128), 0)
    col = lax.broadcasted_iota(jnp.int32, (80, 128), 1)
    idx = row * 128 + col
    real = idx < _N
    key = jnp.where(real, key, jnp.uint32(0))
    k = jnp.int32(_K)
    lo = jnp.uint32(0)
    for b in range(31, -1, -1):
        cand = lo | jnp.uint32(1 << b)
        cnt = jnp.sum((key >= cand).astype(jnp.int32))
        lo = jnp.where(cnt >= k, cand, lo)
    gt = key > lo
    eq = (key == lo) & real
    need = k - jnp.sum(gt.astype(jnp.int32))
    jv = jnp.int32(0)
    for b in range(14, -1, -1):
        cand = jv | jnp.int32(1 << b)
        g = jnp.sum((eq & (idx < cand)).astype(jnp.int32))
        jv = jnp.where(g <= need, cand, jv)
    mask = gt | (eq & (idx < jv))
    mask_ref[...] = mask.astype(jnp.float32)


def _topk(parts):
    return pl.pallas_call(
        _topk_body,
        out_shape=[jax.ShapeDtypeStruct((80, 128), jnp.float32)] * 2,
    )(parts.reshape(2, 80, 128))


def _apply_body(h_ref, c_ref, k_ref, co_ref, ko_ref):
    h = h_ref[...]
    co_ref[...] = h * c_ref[...]
    ko_ref[...] = h * k_ref[...]


def _apply(h2, cw, kw):
    return pl.pallas_call(
        _apply_body,
        grid=(_N // _BLK,),
        in_specs=[
            pl.BlockSpec((_BLK, _D), lambda g: (g, 0)),
            pl.BlockSpec((_BLK, 1), lambda g: (g, 0)),
            pl.BlockSpec((_BLK, 1), lambda g: (g, 0)),
        ],
        out_specs=[
            pl.BlockSpec((_BLK, _D), lambda g: (g, 0)),
            pl.BlockSpec((_BLK, _D), lambda g: (g, 0)),
        ],
        out_shape=[jax.ShapeDtypeStruct((_N, _D), jnp.float32)] * 2,
    )(h2, cw, kw)


@functools.cache
def _mesh():
    return plsc.VectorSubcoreMesh(core_axis_name="c", subcore_axis_name="s")


@functools.cache
def _rowseg_call():
    return functools.partial(
        pl.kernel,
        mesh=_mesh(),
        out_type=[jax.ShapeDtypeStruct((_NP, 64), jnp.float32)] * 4,
        scratch_types=[
            pltpu.VMEM((125, 80), jnp.int32),
            pltpu.VMEM((125, 80), jnp.int32),
            pltpu.VMEM((80, 64), jnp.float32),
            pltpu.VMEM((128, 64), jnp.float32),
            pltpu.VMEM_SHARED((_NP, 64), jnp.float32),
            pltpu.SemaphoreType.DMA,
        ],
        compiler_params=pltpu.CompilerParams(use_tc_tiling_on_sc=False),
    )(_rowseg)


def _rowseg(g0, g1, g2, g3, srcT, dstT, zz, o0, o1, o2, o3, srcbuf, dstbuf,
            rows, obuf, acc, sem):
    c = lax.axis_index("c")
    t = lax.axis_index("s")
    pltpu.sync_copy(srcT.at[t], srcbuf)
    pltpu.sync_copy(dstT.at[t], dstbuf)

    def _round(tbl, oref):
        pltpu.sync_copy(zz, obuf)
        for q in range(5):
            pltpu.sync_copy(obuf, acc.at[pl.ds(t * 640 + q * 128, 128), :])
        plsc.subcore_barrier()

        def body(j, carry):
            pltpu.async_copy(tbl.at[srcbuf.at[j]], rows, sem).wait()
            pltpu.sync_copy(rows, acc.at[dstbuf.at[j]], add=True)
            return carry
        lax.fori_loop(0, 125, body, 0)
        plsc.subcore_barrier()

        for q in range(5):
            pltpu.sync_copy(acc.at[pl.ds(t * 640 + q * 128, 128), :], obuf)
            pltpu.sync_copy(obuf, oref.at[pl.ds(t * 640 + q * 128, 128), :])
        plsc.subcore_barrier()

    @pl.when(c == 0)
    def _():
        _round(g0, o0)
        _round(g1, o1)

    @pl.when(c == 1)
    def _():
        _round(g2, o2)
        _round(g3, o3)


@functools.cache
def _scalseg_call():
    return functools.partial(
        pl.kernel,
        mesh=_mesh(),
        out_type=jax.ShapeDtypeStruct((2, _NP), jnp.float32),
        scratch_types=[
            pltpu.VMEM((40, 128), jnp.int32),
            pltpu.VMEM((40, 128), jnp.int32),
            pltpu.VMEM((128,), jnp.float32),
            pltpu.VMEM((640,), jnp.float32),
            pltpu.VMEM_SHARED((_NP,), jnp.float32),
            pltpu.SemaphoreType.DMA,
        ],
    )(_scalseg)


def _scalseg(s_ext, srcE, dstE, out, srcbuf, dstbuf, valrow, zb, acc, sem):
    c = lax.axis_index("c")
    t = lax.axis_index("s")
    w = c * 16 + t

    def zbody(i, carry):
        zb[pl.ds(i * 16, 16)] = jnp.zeros((16,), jnp.float32)
        return carry
    lax.fori_loop(0, 40, zbody, 0)
    pltpu.sync_copy(zb, acc.at[pl.ds(t * 640, 640)])
    pltpu.sync_copy(srcE.at[w], srcbuf)
    pltpu.sync_copy(dstE.at[w], dstbuf)
    plsc.subcore_barrier()

    def body(j, carry):
        pltpu.async_copy(s_ext.at[srcbuf.at[j]], valrow, sem).wait()
        pltpu.sync_copy(valrow, acc.at[dstbuf.at[j]], add=True)
        return carry
    lax.fori_loop(0, 40, body, 0)

    plsc.subcore_barrier()
    pltpu.sync_copy(acc.at[pl.ds(t * 640, 640)], zb)
    pltpu.sync_copy(zb, out.at[c, pl.ds(t * 640, 640)])


@functools.cache
def _edgemask_call():
    return functools.partial(
        pl.kernel,
        mesh=_mesh(),
        out_type=[jax.ShapeDtypeStruct((32, 40, 128), jnp.float32)] * 2,
        scratch_types=[
            pltpu.VMEM((40, 128), jnp.int32),
            pltpu.VMEM((40, 128), jnp.int32),
            pltpu.VMEM((128,), jnp.float32),
            pltpu.VMEM((128,), jnp.float32),
            pltpu.VMEM((40, 128), jnp.float32),
            pltpu.VMEM((40, 128), jnp.float32),
            pltpu.SemaphoreType.DMA,
            pltpu.SemaphoreType.DMA,
        ],
    )(_edgemask)


def _edgemask(mext, srcE, dstE, ce_out, ke_out, srcbuf, dstbuf, msrow, mdrow,
              cbuf, kbuf, sem, sem2):
    c = lax.axis_index("c")
    t = lax.axis_index("s")
    w = c * 16 + t
    pltpu.sync_copy(srcE.at[w], srcbuf)
    pltpu.sync_copy(dstE.at[w], dstbuf)
    one = jnp.ones((16,), jnp.float32)

    def body(j, carry):
        cps = pltpu.async_copy(mext.at[srcbuf.at[j]], msrow, sem)
        cpd = pltpu.async_copy(mext.at[dstbuf.at[j]], mdrow, sem2)
        cps.wait()
        cpd.wait()
        for u in range(8):
            sl = pl.ds(u * 16, 16)
            ms = msrow[sl]
            md = mdrow[sl]
            cbuf[j, sl] = ms * md
            kbuf[j, sl] = (one - ms) * (one - md)
        return carry
    lax.fori_loop(0, 40, body, 0)
    pltpu.sync_copy(cbuf, ce_out.at[w])
    pltpu.sync_copy(kbuf, ke_out.at[w])


def kernel(x, edge, W1, b1, W2, b2, W3, b3):
    src = edge[0].astype(jnp.int32)
    dst = edge[1].astype(jnp.int32)
    srcT = src.reshape(16, 125, 80)
    dstT = dst.reshape(16, 125, 80)
    pad = jnp.arange(_EPAD - _E, dtype=jnp.int32) % (_NP - _N) + _N
    srcE = jnp.concatenate([src, pad]).reshape(32, 40, 128)
    dstE = jnp.concatenate([dst, pad]).reshape(32, 40, 128)
    zz = jnp.zeros((128, 64), jnp.float32)

    g1q = _mm_split(x, W1, b1, relu_in=False)
    A1 = jnp.concatenate(_rowseg_call()(*g1q, srcT, dstT, zz), axis=1)[:_N]
    g2q = _mm_split(A1, W2, b2, relu_in=True)
    A2 = jnp.concatenate(_rowseg_call()(*g2q, srcT, dstT, zz), axis=1)[:_N]
    h2, s = _mm3(A2, W3, b3)

    s_ext = jnp.concatenate([s[:, 0], jnp.zeros((_NP - _N,), jnp.float32)])
    parts = _scalseg_call()(s_ext, srcE, dstE)
    ns80, mask80 = _topk(parts)
    ns = ns80.reshape(_NP)[:_N]
    node_score = ns[:, None]
    maskv = mask80.reshape(_NP)[:_N]

    w = jax.nn.sigmoid(ns / _TAU)
    cw = (maskv * w)[:, None]
    kw = ((1.0 - maskv) * (1.0 - w))[:, None]
    causal_x, conf_x = _apply(h2, cw, kw)

    mext = jnp.concatenate([maskv, jnp.zeros((_NP - _N,), jnp.float32)])
    ce, ke = _edgemask_call()(mext, srcE, dstE)
    causal_edge = ce.reshape(_EPAD)[:_E]
    conf_edge = ke.reshape(_EPAD)[:_E]
    return (causal_x, causal_edge, conf_x, conf_edge, node_score)


# speedup vs baseline: 9.4555x; 1.5491x over previous
import functools

import jax
import jax.numpy as jnp
from jax import lax
from jax.experimental import pallas as pl
from jax.experimental.pallas import tpu as pltpu
from jax.experimental.pallas import tpu_sc as plsc

_N = 10000
_E = 160000
_D = 256
_K = 5000
_TAU = 0.1
_NP = 10240
_EPAD = 163840
_BLK = 1000


def _mm_split_body(relu_in, x_ref, w_ref, b_ref, o0, o1, o2, o3):
    xb = x_ref[...]
    if relu_in:
        xb = jnp.maximum(xb, 0.0)
    g = jnp.dot(xb, w_ref[...], preferred_element_type=jnp.float32) + b_ref[...]
    o0[...] = g[:, 0:64]
    o1[...] = g[:, 64:128]
    o2[...] = g[:, 128:192]
    o3[...] = g[:, 192:256]


def _mm_split(x, w, b, relu_in):
    return pl.pallas_call(
        functools.partial(_mm_split_body, relu_in),
        grid=(_N // _BLK,),
        in_specs=[
            pl.BlockSpec((_BLK, _D), lambda g: (g, 0)),
            pl.BlockSpec((_D, _D), lambda g: (0, 0)),
            pl.BlockSpec((1, _D), lambda g: (0, 0)),
        ],
        out_specs=[pl.BlockSpec((_BLK, 64), lambda g: (g, 0))] * 4,
        out_shape=[jax.ShapeDtypeStruct((_N, 64), jnp.float32)] * 4,
    )(x, w, b.reshape(1, _D))


def _mm3_body(a_ref, w_ref, b_ref, h_ref, s_ref):
    h = jnp.maximum(a_ref[...], 0.0)
    h_ref[...] = h
    s_ref[...] = jnp.dot(h, w_ref[...], preferred_element_type=jnp.float32) + b_ref[...]


def _mm3(a, w3, b3):
    return pl.pallas_call(
        _mm3_body,
        grid=(_N // _BLK,),
        in_specs=[
            pl.BlockSpec((_BLK, _D), lambda g: (g, 0)),
            pl.BlockSpec((_D, 1), lambda g: (0, 0)),
            pl.BlockSpec((1, 1), lambda g: (0, 0)),
        ],
        out_specs=[
            pl.BlockSpec((_BLK, _D), lambda g: (g, 0)),
            pl.BlockSpec((_BLK, 1), lambda g: (g, 0)),
        ],
        out_shape=[
            jax.ShapeDtypeStruct((_N, _D), jnp.float32),
            jax.ShapeDtypeStruct((_N, 1), jnp.float32),
        ],
    )(a, w3, b3.reshape(1, 1))


def _topk_body(p_ref, ns_ref, mask_ref):
    p = p_ref[0] + p_ref[1]
    ns_ref[...] = p
    bits = lax.bitcast_convert_type(p, jnp.uint32)
    neg = (bits >> jnp.uint32(31)) == jnp.uint32(1)
    key = jnp.where(neg, ~bits, bits | jnp.uint32(0x80000000))
    row = lax.broadcasted_iota(jnp.int32, (80, 128), 0)
    col = lax.broadcasted_iota(jnp.int32, (80, 128), 1)
    idx = row * 128 + col
    real = idx < _N
    key = jnp.where(real, key, jnp.uint32(0))
    k = jnp.int32(_K)
    lo = jnp.uint32(0)
    for b in range(31, -1, -1):
        cand = lo | jnp.uint32(1 << b)
        cnt = jnp.sum((key >= cand).astype(jnp.int32))
        lo = jnp.where(cnt >= k, cand, lo)
    gt = key > lo
    eq = (key == lo) & real
    need = k - jnp.sum(gt.astype(jnp.int32))
    jv = jnp.int32(0)
    for b in range(14, -1, -1):
        cand = jv | jnp.int32(1 << b)
        g = jnp.sum((eq & (idx < cand)).astype(jnp.int32))
        jv = jnp.where(g <= need, cand, jv)
    mask = gt | (eq & (idx < jv))
    mask_ref[...] = mask.astype(jnp.float32)


def _topk(parts):
    return pl.pallas_call(
        _topk_body,
        out_shape=[jax.ShapeDtypeStruct((80, 128), jnp.float32)] * 2,
    )(parts.reshape(2, 80, 128))


def _apply_body(h_ref, c_ref, k_ref, co_ref, ko_ref):
    h = h_ref[...]
    co_ref[...] = h * c_ref[...]
    ko_ref[...] = h * k_ref[...]


def _apply(h2, cw, kw):
    return pl.pallas_call(
        _apply_body,
        grid=(_N // _BLK,),
        in_specs=[
            pl.BlockSpec((_BLK, _D), lambda g: (g, 0)),
            pl.BlockSpec((_BLK, 1), lambda g: (g, 0)),
            pl.BlockSpec((_BLK, 1), lambda g: (g, 0)),
        ],
        out_specs=[
            pl.BlockSpec((_BLK, _D), lambda g: (g, 0)),
            pl.BlockSpec((_BLK, _D), lambda g: (g, 0)),
        ],
        out_shape=[jax.ShapeDtypeStruct((_N, _D), jnp.float32)] * 2,
    )(h2, cw, kw)


@functools.cache
def _mesh():
    return plsc.VectorSubcoreMesh(core_axis_name="c", subcore_axis_name="s")


@functools.cache
def _rowseg_call():
    return functools.partial(
        pl.kernel,
        mesh=_mesh(),
        out_type=[jax.ShapeDtypeStruct((_NP, 64), jnp.float32)] * 4,
        scratch_types=[
            pltpu.VMEM((80, 125), jnp.int32),
            pltpu.VMEM((80, 125), jnp.int32),
            pltpu.VMEM((125, 64), jnp.float32),
            pltpu.VMEM((125, 64), jnp.float32),
            pltpu.VMEM((128, 64), jnp.float32),
            pltpu.VMEM_SHARED((_NP, 64), jnp.float32),
            pltpu.SemaphoreType.DMA,
            pltpu.SemaphoreType.DMA,
        ],
        compiler_params=pltpu.CompilerParams(use_tc_tiling_on_sc=False),
    )(_rowseg)


def _rowseg(g0, g1, g2, g3, srcT, dstT, zz, o0, o1, o2, o3, srcbuf, dstbuf,
            rows, rows2, obuf, acc, sem, sem2):
    c = lax.axis_index("c")
    t = lax.axis_index("s")
    pltpu.sync_copy(srcT.at[t], srcbuf)
    pltpu.sync_copy(dstT.at[t], dstbuf)

    def _round(tbl, oref):
        pltpu.sync_copy(zz, obuf)
        for q in range(5):
            pltpu.sync_copy(obuf, acc.at[pl.ds(t * 640 + q * 128, 128), :])
        plsc.subcore_barrier()

        pltpu.async_copy(tbl.at[srcbuf.at[0]], rows, sem)

        def body(jj, carry):
            j = jj * 2
            pltpu.async_copy(tbl.at[srcbuf.at[j + 1]], rows2, sem2)
            pltpu.make_async_copy(tbl.at[srcbuf.at[0]], rows, sem).wait()
            pltpu.sync_copy(rows, acc.at[dstbuf.at[j]], add=True)
            pltpu.async_copy(tbl.at[srcbuf.at[lax.rem(j + 2, 80)]], rows, sem)
            pltpu.make_async_copy(tbl.at[srcbuf.at[0]], rows2, sem2).wait()
            pltpu.sync_copy(rows2, acc.at[dstbuf.at[j + 1]], add=True)
            return carry
        lax.fori_loop(0, 40, body, 0)
        pltpu.make_async_copy(tbl.at[srcbuf.at[0]], rows, sem).wait()
        plsc.subcore_barrier()

        for q in range(5):
            pltpu.sync_copy(acc.at[pl.ds(t * 640 + q * 128, 128), :], obuf)
            pltpu.sync_copy(obuf, oref.at[pl.ds(t * 640 + q * 128, 128), :])
        plsc.subcore_barrier()

    @pl.when(c == 0)
    def _():
        _round(g0, o0)
        _round(g1, o1)

    @pl.when(c == 1)
    def _():
        _round(g2, o2)
        _round(g3, o3)


@functools.cache
def _scalseg_call():
    return functools.partial(
        pl.kernel,
        mesh=_mesh(),
        out_type=jax.ShapeDtypeStruct((2, _NP), jnp.float32),
        scratch_types=[
            pltpu.VMEM((40, 128), jnp.int32),
            pltpu.VMEM((40, 128), jnp.int32),
            pltpu.VMEM((128,), jnp.float32),
            pltpu.VMEM((640,), jnp.float32),
            pltpu.VMEM_SHARED((_NP,), jnp.float32),
            pltpu.SemaphoreType.DMA,
        ],
    )(_scalseg)


def _scalseg(s_ext, srcE, dstE, out, srcbuf, dstbuf, valrow, zb, acc, sem):
    c = lax.axis_index("c")
    t = lax.axis_index("s")
    w = c * 16 + t

    def zbody(i, carry):
        zb[pl.ds(i * 16, 16)] = jnp.zeros((16,), jnp.float32)
        return carry
    lax.fori_loop(0, 40, zbody, 0)
    pltpu.sync_copy(zb, acc.at[pl.ds(t * 640, 640)])
    pltpu.sync_copy(srcE.at[w], srcbuf)
    pltpu.sync_copy(dstE.at[w], dstbuf)
    plsc.subcore_barrier()

    def body(j, carry):
        pltpu.async_copy(s_ext.at[srcbuf.at[j]], valrow, sem).wait()
        pltpu.sync_copy(valrow, acc.at[dstbuf.at[j]], add=True)
        return carry
    lax.fori_loop(0, 40, body, 0)

    plsc.subcore_barrier()
    pltpu.sync_copy(acc.at[pl.ds(t * 640, 640)], zb)
    pltpu.sync_copy(zb, out.at[c, pl.ds(t * 640, 640)])


@functools.cache
def _edgemask_call():
    return functools.partial(
        pl.kernel,
        mesh=_mesh(),
        out_type=[jax.ShapeDtypeStruct((32, 40, 128), jnp.float32)] * 2,
        scratch_types=[
            pltpu.VMEM((40, 128), jnp.int32),
            pltpu.VMEM((40, 128), jnp.int32),
            pltpu.VMEM((128,), jnp.float32),
            pltpu.VMEM((128,), jnp.float32),
            pltpu.VMEM((40, 128), jnp.float32),
            pltpu.VMEM((40, 128), jnp.float32),
            pltpu.SemaphoreType.DMA,
            pltpu.SemaphoreType.DMA,
        ],
    )(_edgemask)


def _edgemask(mext, srcE, dstE, ce_out, ke_out, srcbuf, dstbuf, msrow, mdrow,
              cbuf, kbuf, sem, sem2):
    c = lax.axis_index("c")
    t = lax.axis_index("s")
    w = c * 16 + t
    pltpu.sync_copy(srcE.at[w], srcbuf)
    pltpu.sync_copy(dstE.at[w], dstbuf)
    one = jnp.ones((16,), jnp.float32)

    def body(j, carry):
        cps = pltpu.async_copy(mext.at[srcbuf.at[j]], msrow, sem)
        cpd = pltpu.async_copy(mext.at[dstbuf.at[j]], mdrow, sem2)
        cps.wait()
        cpd.wait()
        for u in range(8):
            sl = pl.ds(u * 16, 16)
            ms = msrow[sl]
            md = mdrow[sl]
            cbuf[j, sl] = ms * md
            kbuf[j, sl] = (one - ms) * (one - md)
        return carry
    lax.fori_loop(0, 40, body, 0)
    pltpu.sync_copy(cbuf, ce_out.at[w])
    pltpu.sync_copy(kbuf, ke_out.at[w])


def kernel(x, edge, W1, b1, W2, b2, W3, b3):
    src = edge[0].astype(jnp.int32)
    dst = edge[1].astype(jnp.int32)
    srcT = src.reshape(16, 80, 125)
    dstT = dst.reshape(16, 80, 125)
    pad = jnp.arange(_EPAD - _E, dtype=jnp.int32) % (_NP - _N) + _N
    srcE = jnp.concatenate([src, pad]).reshape(32, 40, 128)
    dstE = jnp.concatenate([dst, pad]).reshape(32, 40, 128)
    zz = jnp.zeros((128, 64), jnp.float32)

    g1q = _mm_split(x, W1, b1, relu_in=False)
    A1 = jnp.concatenate(_rowseg_call()(*g1q, srcT, dstT, zz), axis=1)[:_N]
    g2q = _mm_split(A1, W2, b2, relu_in=True)
    A2 = jnp.concatenate(_rowseg_call()(*g2q, srcT, dstT, zz), axis=1)[:_N]
    h2, s = _mm3(A2, W3, b3)

    s_ext = jnp.concatenate([s[:, 0], jnp.zeros((_NP - _N,), jnp.float32)])
    parts = _scalseg_call()(s_ext, srcE, dstE)
    ns80, mask80 = _topk(parts)
    ns = ns80.reshape(_NP)[:_N]
    node_score = ns[:, None]
    maskv = mask80.reshape(_NP)[:_N]

    w = jax.nn.sigmoid(ns / _TAU)
    cw = (maskv * w)[:, None]
    kw = ((1.0 - maskv) * (1.0 - w))[:, None]
    causal_x, conf_x = _apply(h2, cw, kw)

    mext = jnp.concatenate([maskv, jnp.zeros((_NP - _N,), jnp.float32)])
    ce, ke = _edgemask_call()(mext, srcE, dstE)
    causal_edge = ce.reshape(_EPAD)[:_E]
    conf_edge = ke.reshape(_EPAD)[:_E]
    return (causal_x, causal_edge, conf_x, conf_edge, node_score)


# speedup vs baseline: 9.9246x; 1.0496x over previous
import functools

import jax
import jax.numpy as jnp
from jax import lax
from jax.experimental import pallas as pl
from jax.experimental.pallas import tpu as pltpu
from jax.experimental.pallas import tpu_sc as plsc

_N = 10000
_E = 160000
_D = 256
_K = 5000
_TAU = 0.1
_NP = 10240
_EPAD = 163840
_BLK = 1000


def _mm_split_body(relu_in, x_ref, w_ref, b_ref, o0, o1, o2, o3):
    xb = x_ref[...]
    if relu_in:
        xb = jnp.maximum(xb, 0.0)
    g = jnp.dot(xb, w_ref[...], preferred_element_type=jnp.float32) + b_ref[...]
    o0[...] = g[:, 0:64]
    o1[...] = g[:, 64:128]
    o2[...] = g[:, 128:192]
    o3[...] = g[:, 192:256]


def _mm_split(x, w, b, relu_in):
    return pl.pallas_call(
        functools.partial(_mm_split_body, relu_in),
        grid=(_N // _BLK,),
        in_specs=[
            pl.BlockSpec((_BLK, _D), lambda g: (g, 0)),
            pl.BlockSpec((_D, _D), lambda g: (0, 0)),
            pl.BlockSpec((1, _D), lambda g: (0, 0)),
        ],
        out_specs=[pl.BlockSpec((_BLK, 64), lambda g: (g, 0))] * 4,
        out_shape=[jax.ShapeDtypeStruct((_N, 64), jnp.float32)] * 4,
    )(x, w, b.reshape(1, _D))


def _mm3_body(a_ref, w_ref, b_ref, h_ref, s_ref):
    h = jnp.maximum(a_ref[...], 0.0)
    h_ref[...] = h
    s_ref[...] = jnp.dot(h, w_ref[...], preferred_element_type=jnp.float32) + b_ref[...]


def _mm3(a, w3, b3):
    return pl.pallas_call(
        _mm3_body,
        grid=(_N // _BLK,),
        in_specs=[
            pl.BlockSpec((_BLK, _D), lambda g: (g, 0)),
            pl.BlockSpec((_D, 1), lambda g: (0, 0)),
            pl.BlockSpec((1, 1), lambda g: (0, 0)),
        ],
        out_specs=[
            pl.BlockSpec((_BLK, _D), lambda g: (g, 0)),
            pl.BlockSpec((_BLK, 1), lambda g: (g, 0)),
        ],
        out_shape=[
            jax.ShapeDtypeStruct((_N, _D), jnp.float32),
            jax.ShapeDtypeStruct((_N, 1), jnp.float32),
        ],
    )(a, w3, b3.reshape(1, 1))


def _topk_body(p_ref, ns_ref, mask_ref):
    p = p_ref[0] + p_ref[1]
    ns_ref[...] = p
    bits = lax.bitcast_convert_type(p, jnp.uint32)
    neg = (bits >> jnp.uint32(31)) == jnp.uint32(1)
    key = jnp.where(neg, ~bits, bits | jnp.uint32(0x80000000))
    row = lax.broadcasted_iota(jnp.int32, (80, 128), 0)
    col = lax.broadcasted_iota(jnp.int32, (80, 128), 1)
    idx = row * 128 + col
    real = idx < _N
    key = jnp.where(real, key, jnp.uint32(0))
    k = jnp.int32(_K)
    lo = jnp.uint32(0)
    for b in range(31, -1, -1):
        cand = lo | jnp.uint32(1 << b)
        cnt = jnp.sum((key >= cand).astype(jnp.int32))
        lo = jnp.where(cnt >= k, cand, lo)
    gt = key > lo
    eq = (key == lo) & real
    need = k - jnp.sum(gt.astype(jnp.int32))
    jv = jnp.int32(0)
    for b in range(14, -1, -1):
        cand = jv | jnp.int32(1 << b)
        g = jnp.sum((eq & (idx < cand)).astype(jnp.int32))
        jv = jnp.where(g <= need, cand, jv)
    mask = gt | (eq & (idx < jv))
    mask_ref[...] = mask.astype(jnp.float32)


def _topk(parts):
    return pl.pallas_call(
        _topk_body,
        out_shape=[jax.ShapeDtypeStruct((80, 128), jnp.float32)] * 2,
    )(parts.reshape(2, 80, 128))


def _apply_body(h_ref, c_ref, k_ref, co_ref, ko_ref):
    h = h_ref[...]
    co_ref[...] = h * c_ref[...]
    ko_ref[...] = h * k_ref[...]


def _apply(h2, cw, kw):
    return pl.pallas_call(
        _apply_body,
        grid=(_N // _BLK,),
        in_specs=[
            pl.BlockSpec((_BLK, _D), lambda g: (g, 0)),
            pl.BlockSpec((_BLK, 1), lambda g: (g, 0)),
            pl.BlockSpec((_BLK, 1), lambda g: (g, 0)),
        ],
        out_specs=[
            pl.BlockSpec((_BLK, _D), lambda g: (g, 0)),
            pl.BlockSpec((_BLK, _D), lambda g: (g, 0)),
        ],
        out_shape=[jax.ShapeDtypeStruct((_N, _D), jnp.float32)] * 2,
    )(h2, cw, kw)


@functools.cache
def _mesh():
    return plsc.VectorSubcoreMesh(core_axis_name="c", subcore_axis_name="s")


@functools.cache
def _rowseg_call():
    return functools.partial(
        pl.kernel,
        mesh=_mesh(),
        out_type=[jax.ShapeDtypeStruct((_NP, 64), jnp.float32)] * 4,
        scratch_types=[
            pltpu.VMEM((80, 125), jnp.int32),
            pltpu.VMEM((80, 125), jnp.int32),
            pltpu.VMEM((125, 64), jnp.float32),
            pltpu.VMEM((125, 64), jnp.float32),
            pltpu.VMEM((128, 64), jnp.float32),
            pltpu.VMEM_SHARED((_NP, 64), jnp.float32),
            pltpu.SemaphoreType.DMA,
            pltpu.SemaphoreType.DMA,
        ],
        compiler_params=pltpu.CompilerParams(use_tc_tiling_on_sc=False),
    )(_rowseg)


def _rowseg(g0, g1, g2, g3, srcT, dstT, zz, o0, o1, o2, o3, srcbuf, dstbuf,
            rows, rows2, obuf, acc, sem, sem2):
    c = lax.axis_index("c")
    t = lax.axis_index("s")
    pltpu.sync_copy(srcT.at[t], srcbuf)
    pltpu.sync_copy(dstT.at[t], dstbuf)

    def _round(tbl, oref):
        pltpu.sync_copy(zz, obuf)
        for q in range(5):
            pltpu.sync_copy(obuf, acc.at[pl.ds(t * 640 + q * 128, 128), :])
        plsc.subcore_barrier()

        pltpu.async_copy(tbl.at[srcbuf.at[0]], rows, sem)

        def body(jj, carry):
            j = jj * 2
            pltpu.async_copy(tbl.at[srcbuf.at[j + 1]], rows2, sem2)
            pltpu.make_async_copy(tbl.at[srcbuf.at[0]], rows, sem).wait()
            pltpu.sync_copy(rows, acc.at[dstbuf.at[j]], add=True)
            pltpu.async_copy(tbl.at[srcbuf.at[lax.rem(j + 2, 80)]], rows, sem)
            pltpu.make_async_copy(tbl.at[srcbuf.at[0]], rows2, sem2).wait()
            pltpu.sync_copy(rows2, acc.at[dstbuf.at[j + 1]], add=True)
            return carry
        lax.fori_loop(0, 40, body, 0)
        pltpu.make_async_copy(tbl.at[srcbuf.at[0]], rows, sem).wait()
        plsc.subcore_barrier()

        pltpu.sync_copy(acc.at[pl.ds(t * 640, 640), :],
                        oref.at[pl.ds(t * 640, 640), :])
        plsc.subcore_barrier()

    @pl.when(c == 0)
    def _():
        _round(g0, o0)
        _round(g1, o1)

    @pl.when(c == 1)
    def _():
        _round(g2, o2)
        _round(g3, o3)


@functools.cache
def _scalseg_call():
    return functools.partial(
        pl.kernel,
        mesh=_mesh(),
        out_type=jax.ShapeDtypeStruct((2, _NP), jnp.float32),
        scratch_types=[
            pltpu.VMEM((40, 128), jnp.int32),
            pltpu.VMEM((40, 128), jnp.int32),
            pltpu.VMEM((128,), jnp.float32),
            pltpu.VMEM((128,), jnp.float32),
            pltpu.VMEM((640,), jnp.float32),
            pltpu.VMEM_SHARED((_NP,), jnp.float32),
            pltpu.SemaphoreType.DMA,
            pltpu.SemaphoreType.DMA,
        ],
    )(_scalseg)


def _scalseg(s_ext, srcE, dstE, out, srcbuf, dstbuf, valrow, valrow2, zb, acc,
             sem, sem2):
    c = lax.axis_index("c")
    t = lax.axis_index("s")
    w = c * 16 + t

    def zbody(i, carry):
        zb[pl.ds(i * 16, 16)] = jnp.zeros((16,), jnp.float32)
        return carry
    lax.fori_loop(0, 40, zbody, 0)
    pltpu.sync_copy(zb, acc.at[pl.ds(t * 640, 640)])
    pltpu.sync_copy(srcE.at[w], srcbuf)
    pltpu.sync_copy(dstE.at[w], dstbuf)
    plsc.subcore_barrier()

    pltpu.async_copy(s_ext.at[srcbuf.at[0]], valrow, sem)

    def body(jj, carry):
        j = jj * 2
        pltpu.async_copy(s_ext.at[srcbuf.at[j + 1]], valrow2, sem2)
        pltpu.make_async_copy(s_ext.at[srcbuf.at[0]], valrow, sem).wait()
        pltpu.sync_copy(valrow, acc.at[dstbuf.at[j]], add=True)
        pltpu.async_copy(s_ext.at[srcbuf.at[lax.rem(j + 2, 40)]], valrow, sem)
        pltpu.make_async_copy(s_ext.at[srcbuf.at[0]], valrow2, sem2).wait()
        pltpu.sync_copy(valrow2, acc.at[dstbuf.at[j + 1]], add=True)
        return carry
    lax.fori_loop(0, 20, body, 0)
    pltpu.make_async_copy(s_ext.at[srcbuf.at[0]], valrow, sem).wait()

    plsc.subcore_barrier()
    pltpu.sync_copy(acc.at[pl.ds(t * 640, 640)], zb)
    pltpu.sync_copy(zb, out.at[c, pl.ds(t * 640, 640)])


@functools.cache
def _edgemask_call():
    return functools.partial(
        pl.kernel,
        mesh=_mesh(),
        out_type=[jax.ShapeDtypeStruct((32, 40, 128), jnp.float32)] * 2,
        scratch_types=[
            pltpu.VMEM((40, 128), jnp.int32),
            pltpu.VMEM((40, 128), jnp.int32),
            pltpu.VMEM((128,), jnp.float32),
            pltpu.VMEM((128,), jnp.float32),
            pltpu.VMEM((128,), jnp.float32),
            pltpu.VMEM((128,), jnp.float32),
            pltpu.VMEM((40, 128), jnp.float32),
            pltpu.VMEM((40, 128), jnp.float32),
            pltpu.SemaphoreType.DMA,
            pltpu.SemaphoreType.DMA,
            pltpu.SemaphoreType.DMA,
            pltpu.SemaphoreType.DMA,
        ],
    )(_edgemask)


def _edgemask(mext, srcE, dstE, ce_out, ke_out, srcbuf, dstbuf, msrow, mdrow,
              msrow2, mdrow2, cbuf, kbuf, sem, sem2, sem3, sem4):
    c = lax.axis_index("c")
    t = lax.axis_index("s")
    w = c * 16 + t
    pltpu.sync_copy(srcE.at[w], srcbuf)
    pltpu.sync_copy(dstE.at[w], dstbuf)
    one = jnp.ones((16,), jnp.float32)

    def compute(j, ms_ref, md_ref):
        for u in range(8):
            sl = pl.ds(u * 16, 16)
            ms = ms_ref[sl]
            md = md_ref[sl]
            cbuf[j, sl] = ms * md
            kbuf[j, sl] = (one - ms) * (one - md)

    pltpu.async_copy(mext.at[srcbuf.at[0]], msrow, sem)
    pltpu.async_copy(mext.at[dstbuf.at[0]], mdrow, sem2)

    def body(jj, carry):
        j = jj * 2
        pltpu.async_copy(mext.at[srcbuf.at[j + 1]], msrow2, sem3)
        pltpu.async_copy(mext.at[dstbuf.at[j + 1]], mdrow2, sem4)
        pltpu.make_async_copy(mext.at[srcbuf.at[0]], msrow, sem).wait()
        pltpu.make_async_copy(mext.at[dstbuf.at[0]], mdrow, sem2).wait()
        compute(j, msrow, mdrow)
        jn = lax.rem(j + 2, 40)
        pltpu.async_copy(mext.at[srcbuf.at[jn]], msrow, sem)
        pltpu.async_copy(mext.at[dstbuf.at[jn]], mdrow, sem2)
        pltpu.make_async_copy(mext.at[srcbuf.at[0]], msrow2, sem3).wait()
        pltpu.make_async_copy(mext.at[dstbuf.at[0]], mdrow2, sem4).wait()
        compute(j + 1, msrow2, mdrow2)
        return carry
    lax.fori_loop(0, 20, body, 0)
    pltpu.make_async_copy(mext.at[srcbuf.at[0]], msrow, sem).wait()
    pltpu.make_async_copy(mext.at[dstbuf.at[0]], mdrow, sem2).wait()
    pltpu.sync_copy(cbuf, ce_out.at[w])
    pltpu.sync_copy(kbuf, ke_out.at[w])


def kernel(x, edge, W1, b1, W2, b2, W3, b3):
    src = edge[0].astype(jnp.int32)
    dst = edge[1].astype(jnp.int32)
    srcT = src.reshape(16, 80, 125)
    dstT = dst.reshape(16, 80, 125)
    pad = jnp.arange(_EPAD - _E, dtype=jnp.int32) % (_NP - _N) + _N
    srcE = jnp.concatenate([src, pad]).reshape(32, 40, 128)
    dstE = jnp.concatenate([dst, pad]).reshape(32, 40, 128)
    zz = jnp.zeros((128, 64), jnp.float32)

    g1q = _mm_split(x, W1, b1, relu_in=False)
    A1 = jnp.concatenate(_rowseg_call()(*g1q, srcT, dstT, zz), axis=1)[:_N]
    g2q = _mm_split(A1, W2, b2, relu_in=True)
    A2 = jnp.concatenate(_rowseg_call()(*g2q, srcT, dstT, zz), axis=1)[:_N]
    h2, s = _mm3(A2, W3, b3)

    s_ext = jnp.concatenate([s[:, 0], jnp.zeros((_NP - _N,), jnp.float32)])
    parts = _scalseg_call()(s_ext, srcE, dstE)
    ns80, mask80 = _topk(parts)
    ns = ns80.reshape(_NP)[:_N]
    node_score = ns[:, None]
    maskv = mask80.reshape(_NP)[:_N]

    w = jax.nn.sigmoid(ns / _TAU)
    cw = (maskv * w)[:, None]
    kw = ((1.0 - maskv) * (1.0 - w))[:, None]
    causal_x, conf_x = _apply(h2, cw, kw)

    mext = jnp.concatenate([maskv, jnp.zeros((_NP - _N,), jnp.float32)])
    ce, ke = _edgemask_call()(mext, srcE, dstE)
    causal_edge = ce.reshape(_EPAD)[:_E]
    conf_edge = ke.reshape(_EPAD)[:_E]
    return (causal_x, causal_edge, conf_x, conf_edge, node_score)


# speedup vs baseline: 10.3308x; 1.0409x over previous
import functools

import jax
import jax.numpy as jnp
from jax import lax
from jax.experimental import pallas as pl
from jax.experimental.pallas import tpu as pltpu
from jax.experimental.pallas import tpu_sc as plsc

_N = 10000
_E = 160000
_D = 256
_K = 5000
_TAU = 0.1
_NP = 10240
_EPAD = 163840
_BLK = 1000


def _mm_split_body(relu_in, x_ref, w_ref, b_ref, o0, o1, o2, o3):
    xb = x_ref[...]
    if relu_in:
        xb = jnp.maximum(xb, 0.0)
    g = jnp.dot(xb, w_ref[...], preferred_element_type=jnp.float32) + b_ref[...]
    o0[...] = g[:, 0:64]
    o1[...] = g[:, 64:128]
    o2[...] = g[:, 128:192]
    o3[...] = g[:, 192:256]


def _mm_split(x, w, b, relu_in):
    return pl.pallas_call(
        functools.partial(_mm_split_body, relu_in),
        grid=(_N // _BLK,),
        in_specs=[
            pl.BlockSpec((_BLK, _D), lambda g: (g, 0)),
            pl.BlockSpec((_D, _D), lambda g: (0, 0)),
            pl.BlockSpec((1, _D), lambda g: (0, 0)),
        ],
        out_specs=[pl.BlockSpec((_BLK, 64), lambda g: (g, 0))] * 4,
        out_shape=[jax.ShapeDtypeStruct((_N, 64), jnp.float32)] * 4,
    )(x, w, b.reshape(1, _D))


def _mm_splitq_body(x_ref0, x_ref1, x_ref2, x_ref3, w_ref, b_ref, o0, o1, o2, o3):
    xb = jnp.maximum(jnp.concatenate(
        [x_ref0[...], x_ref1[...], x_ref2[...], x_ref3[...]], axis=1), 0.0)
    g = jnp.dot(xb, w_ref[...], preferred_element_type=jnp.float32) + b_ref[...]
    o0[...] = g[:, 0:64]
    o1[...] = g[:, 64:128]
    o2[...] = g[:, 128:192]
    o3[...] = g[:, 192:256]


def _mm_splitq(q0, q1, q2, q3, w, b):
    return pl.pallas_call(
        _mm_splitq_body,
        grid=(_N // _BLK,),
        in_specs=[pl.BlockSpec((_BLK, 64), lambda g: (g, 0))] * 4 + [
            pl.BlockSpec((_D, _D), lambda g: (0, 0)),
            pl.BlockSpec((1, _D), lambda g: (0, 0)),
        ],
        out_specs=[pl.BlockSpec((_BLK, 64), lambda g: (g, 0))] * 4,
        out_shape=[jax.ShapeDtypeStruct((_N, 64), jnp.float32)] * 4,
    )(q0, q1, q2, q3, w, b.reshape(1, _D))


def _mm3_body(x_ref0, x_ref1, x_ref2, x_ref3, w_ref, b_ref, h_ref, s_ref):
    h = jnp.maximum(jnp.concatenate(
        [x_ref0[...], x_ref1[...], x_ref2[...], x_ref3[...]], axis=1), 0.0)
    h_ref[...] = h
    s_ref[...] = jnp.dot(h, w_ref[...], preferred_element_type=jnp.float32) + b_ref[...]


def _mm3(q0, q1, q2, q3, w3, b3):
    return pl.pallas_call(
        _mm3_body,
        grid=(_N // _BLK,),
        in_specs=[pl.BlockSpec((_BLK, 64), lambda g: (g, 0))] * 4 + [
            pl.BlockSpec((_D, 1), lambda g: (0, 0)),
            pl.BlockSpec((1, 1), lambda g: (0, 0)),
        ],
        out_specs=[
            pl.BlockSpec((_BLK, _D), lambda g: (g, 0)),
            pl.BlockSpec((_BLK, 1), lambda g: (g, 0)),
        ],
        out_shape=[
            jax.ShapeDtypeStruct((_N, _D), jnp.float32),
            jax.ShapeDtypeStruct((_N, 1), jnp.float32),
        ],
    )(q0, q1, q2, q3, w3, b3.reshape(1, 1))


def _topk_body(p_ref, ns_ref, mask_ref):
    p = p_ref[0] + p_ref[1]
    ns_ref[...] = p
    bits = lax.bitcast_convert_type(p, jnp.uint32)
    neg = (bits >> jnp.uint32(31)) == jnp.uint32(1)
    key = jnp.where(neg, ~bits, bits | jnp.uint32(0x80000000))
    row = lax.broadcasted_iota(jnp.int32, (80, 128), 0)
    col = lax.broadcasted_iota(jnp.int32, (80, 128), 1)
    idx = row * 128 + col
    real = idx < _N
    key = jnp.where(real, key, jnp.uint32(0))
    k = jnp.int32(_K)
    lo = jnp.uint32(0)
    for b in range(31, -1, -1):
        cand = lo | jnp.uint32(1 << b)
        cnt = jnp.sum((key >= cand).astype(jnp.int32))
        lo = jnp.where(cnt >= k, cand, lo)
    gt = key > lo
    eq = (key == lo) & real
    need = k - jnp.sum(gt.astype(jnp.int32))
    jv = jnp.int32(0)
    for b in range(14, -1, -1):
        cand = jv | jnp.int32(1 << b)
        g = jnp.sum((eq & (idx < cand)).astype(jnp.int32))
        jv = jnp.where(g <= need, cand, jv)
    mask = gt | (eq & (idx < jv))
    mask_ref[...] = mask.astype(jnp.float32)


def _topk(parts):
    return pl.pallas_call(
        _topk_body,
        out_shape=[jax.ShapeDtypeStruct((80, 128), jnp.float32)] * 2,
    )(parts.reshape(2, 80, 128))


def _apply_body(h_ref, c_ref, k_ref, co_ref, ko_ref):
    h = h_ref[...]
    co_ref[...] = h * c_ref[...]
    ko_ref[...] = h * k_ref[...]


def _apply(h2, cw, kw):
    return pl.pallas_call(
        _apply_body,
        grid=(_N // _BLK,),
        in_specs=[
            pl.BlockSpec((_BLK, _D), lambda g: (g, 0)),
            pl.BlockSpec((_BLK, 1), lambda g: (g, 0)),
            pl.BlockSpec((_BLK, 1), lambda g: (g, 0)),
        ],
        out_specs=[
            pl.BlockSpec((_BLK, _D), lambda g: (g, 0)),
            pl.BlockSpec((_BLK, _D), lambda g: (g, 0)),
        ],
        out_shape=[jax.ShapeDtypeStruct((_N, _D), jnp.float32)] * 2,
    )(h2, cw, kw)


@functools.cache
def _mesh():
    return plsc.VectorSubcoreMesh(core_axis_name="c", subcore_axis_name="s")


@functools.cache
def _rowseg_call():
    return functools.partial(
        pl.kernel,
        mesh=_mesh(),
        out_type=[jax.ShapeDtypeStruct((_NP, 64), jnp.float32)] * 4,
        scratch_types=[
            pltpu.VMEM((80, 125), jnp.int32),
            pltpu.VMEM((80, 125), jnp.int32),
            pltpu.VMEM((125, 64), jnp.float32),
            pltpu.VMEM((125, 64), jnp.float32),
            pltpu.VMEM((128, 64), jnp.float32),
            pltpu.VMEM_SHARED((_NP, 64), jnp.float32),
            pltpu.SemaphoreType.DMA,
            pltpu.SemaphoreType.DMA,
        ],
        compiler_params=pltpu.CompilerParams(use_tc_tiling_on_sc=False),
    )(_rowseg)


def _rowseg(g0, g1, g2, g3, srcT, dstT, zz, o0, o1, o2, o3, srcbuf, dstbuf,
            rows, rows2, obuf, acc, sem, sem2):
    c = lax.axis_index("c")
    t = lax.axis_index("s")
    pltpu.sync_copy(srcT.at[t], srcbuf)
    pltpu.sync_copy(dstT.at[t], dstbuf)

    def _round(tbl, oref):
        pltpu.sync_copy(zz, obuf)
        for q in range(5):
            pltpu.sync_copy(obuf, acc.at[pl.ds(t * 640 + q * 128, 128), :])
        plsc.subcore_barrier()

        pltpu.async_copy(tbl.at[srcbuf.at[0]], rows, sem)

        def body(jj, carry):
            j = jj * 2
            pltpu.async_copy(tbl.at[srcbuf.at[j + 1]], rows2, sem2)
            pltpu.make_async_copy(tbl.at[srcbuf.at[0]], rows, sem).wait()
            pltpu.sync_copy(rows, acc.at[dstbuf.at[j]], add=True)
            pltpu.async_copy(tbl.at[srcbuf.at[lax.rem(j + 2, 80)]], rows, sem)
            pltpu.make_async_copy(tbl.at[srcbuf.at[0]], rows2, sem2).wait()
            pltpu.sync_copy(rows2, acc.at[dstbuf.at[j + 1]], add=True)
            return carry
        lax.fori_loop(0, 40, body, 0)
        pltpu.make_async_copy(tbl.at[srcbuf.at[0]], rows, sem).wait()
        plsc.subcore_barrier()

        pltpu.sync_copy(acc.at[pl.ds(t * 640, 640), :],
                        oref.at[pl.ds(t * 640, 640), :])
        plsc.subcore_barrier()

    @pl.when(c == 0)
    def _():
        _round(g0, o0)
        _round(g1, o1)

    @pl.when(c == 1)
    def _():
        _round(g2, o2)
        _round(g3, o3)


@functools.cache
def _scalseg_call():
    return functools.partial(
        pl.kernel,
        mesh=_mesh(),
        out_type=jax.ShapeDtypeStruct((2, _NP), jnp.float32),
        scratch_types=[
            pltpu.VMEM((40, 128), jnp.int32),
            pltpu.VMEM((40, 128), jnp.int32),
            pltpu.VMEM((128,), jnp.float32),
            pltpu.VMEM((128,), jnp.float32),
            pltpu.VMEM((640,), jnp.float32),
            pltpu.VMEM_SHARED((_NP,), jnp.float32),
            pltpu.SemaphoreType.DMA,
            pltpu.SemaphoreType.DMA,
        ],
    )(_scalseg)


def _scalseg(s_ext, srcE, dstE, out, srcbuf, dstbuf, valrow, valrow2, zb, acc,
             sem, sem2):
    c = lax.axis_index("c")
    t = lax.axis_index("s")
    w = c * 16 + t

    def zbody(i, carry):
        zb[pl.ds(i * 16, 16)] = jnp.zeros((16,), jnp.float32)
        return carry
    lax.fori_loop(0, 40, zbody, 0)
    pltpu.sync_copy(zb, acc.at[pl.ds(t * 640, 640)])
    pltpu.sync_copy(srcE.at[w], srcbuf)
    pltpu.sync_copy(dstE.at[w], dstbuf)
    plsc.subcore_barrier()

    pltpu.async_copy(s_ext.at[srcbuf.at[0]], valrow, sem)

    def body(jj, carry):
        j = jj * 2
        pltpu.async_copy(s_ext.at[srcbuf.at[j + 1]], valrow2, sem2)
        pltpu.make_async_copy(s_ext.at[srcbuf.at[0]], valrow, sem).wait()
        pltpu.sync_copy(valrow, acc.at[dstbuf.at[j]], add=True)
        pltpu.async_copy(s_ext.at[srcbuf.at[lax.rem(j + 2, 40)]], valrow, sem)
        pltpu.make_async_copy(s_ext.at[srcbuf.at[0]], valrow2, sem2).wait()
        pltpu.sync_copy(valrow2, acc.at[dstbuf.at[j + 1]], add=True)
        return carry
    lax.fori_loop(0, 20, body, 0)
    pltpu.make_async_copy(s_ext.at[srcbuf.at[0]], valrow, sem).wait()

    plsc.subcore_barrier()
    pltpu.sync_copy(acc.at[pl.ds(t * 640, 640)], zb)
    pltpu.sync_copy(zb, out.at[c, pl.ds(t * 640, 640)])


@functools.cache
def _edgemask_call():
    return functools.partial(
        pl.kernel,
        mesh=_mesh(),
        out_type=[jax.ShapeDtypeStruct((32, 40, 128), jnp.float32)] * 2,
        scratch_types=[
            pltpu.VMEM((40, 128), jnp.int32),
            pltpu.VMEM((40, 128), jnp.int32),
            pltpu.VMEM((128,), jnp.float32),
            pltpu.VMEM((128,), jnp.float32),
            pltpu.VMEM((128,), jnp.float32),
            pltpu.VMEM((128,), jnp.float32),
            pltpu.VMEM((40, 128), jnp.float32),
            pltpu.VMEM((40, 128), jnp.float32),
            pltpu.SemaphoreType.DMA,
            pltpu.SemaphoreType.DMA,
            pltpu.SemaphoreType.DMA,
            pltpu.SemaphoreType.DMA,
        ],
    )(_edgemask)


def _edgemask(mext, srcE, dstE, ce_out, ke_out, srcbuf, dstbuf, msrow, mdrow,
              msrow2, mdrow2, cbuf, kbuf, sem, sem2, sem3, sem4):
    c = lax.axis_index("c")
    t = lax.axis_index("s")
    w = c * 16 + t
    pltpu.sync_copy(srcE.at[w], srcbuf)
    pltpu.sync_copy(dstE.at[w], dstbuf)
    one = jnp.ones((16,), jnp.float32)

    def compute(j, ms_ref, md_ref):
        for u in range(8):
            sl = pl.ds(u * 16, 16)
            ms = ms_ref[sl]
            md = md_ref[sl]
            cbuf[j, sl] = ms * md
            kbuf[j, sl] = (one - ms) * (one - md)

    pltpu.async_copy(mext.at[srcbuf.at[0]], msrow, sem)
    pltpu.async_copy(mext.at[dstbuf.at[0]], mdrow, sem2)

    def body(jj, carry):
        j = jj * 2
        pltpu.async_copy(mext.at[srcbuf.at[j + 1]], msrow2, sem3)
        pltpu.async_copy(mext.at[dstbuf.at[j + 1]], mdrow2, sem4)
        pltpu.make_async_copy(mext.at[srcbuf.at[0]], msrow, sem).wait()
        pltpu.make_async_copy(mext.at[dstbuf.at[0]], mdrow, sem2).wait()
        compute(j, msrow, mdrow)
        jn = lax.rem(j + 2, 40)
        pltpu.async_copy(mext.at[srcbuf.at[jn]], msrow, sem)
        pltpu.async_copy(mext.at[dstbuf.at[jn]], mdrow, sem2)
        pltpu.make_async_copy(mext.at[srcbuf.at[0]], msrow2, sem3).wait()
        pltpu.make_async_copy(mext.at[dstbuf.at[0]], mdrow2, sem4).wait()
        compute(j + 1, msrow2, mdrow2)
        return carry
    lax.fori_loop(0, 20, body, 0)
    pltpu.make_async_copy(mext.at[srcbuf.at[0]], msrow, sem).wait()
    pltpu.make_async_copy(mext.at[dstbuf.at[0]], mdrow, sem2).wait()
    pltpu.sync_copy(cbuf, ce_out.at[w])
    pltpu.sync_copy(kbuf, ke_out.at[w])


def kernel(x, edge, W1, b1, W2, b2, W3, b3):
    src = edge[0].astype(jnp.int32)
    dst = edge[1].astype(jnp.int32)
    srcT = src.reshape(16, 80, 125)
    dstT = dst.reshape(16, 80, 125)
    pad = jnp.arange(_EPAD - _E, dtype=jnp.int32) % (_NP - _N) + _N
    srcE = jnp.concatenate([src, pad]).reshape(32, 40, 128)
    dstE = jnp.concatenate([dst, pad]).reshape(32, 40, 128)
    zz = jnp.zeros((128, 64), jnp.float32)

    g1q = _mm_split(x, W1, b1, relu_in=False)
    A1q = _rowseg_call()(*g1q, srcT, dstT, zz)
    g2q = _mm_splitq(*A1q, W2, b2)
    A2q = _rowseg_call()(*g2q, srcT, dstT, zz)
    h2, s = _mm3(*A2q, W3, b3)

    s_ext = jnp.concatenate([s[:, 0], jnp.zeros((_NP - _N,), jnp.float32)])
    parts = _scalseg_call()(s_ext, srcE, dstE)
    ns80, mask80 = _topk(parts)
    ns = ns80.reshape(_NP)[:_N]
    node_score = ns[:, None]
    maskv = mask80.reshape(_NP)[:_N]

    w = jax.nn.sigmoid(ns / _TAU)
    cw = (maskv * w)[:, None]
    kw = ((1.0 - maskv) * (1.0 - w))[:, None]
    causal_x, conf_x = _apply(h2, cw, kw)

    mext = jnp.concatenate([maskv, jnp.zeros((_NP - _N,), jnp.float32)])
    ce, ke = _edgemask_call()(mext, srcE, dstE)
    causal_edge = ce.reshape(_EPAD)[:_E]
    conf_edge = ke.reshape(_EPAD)[:_E]
    return (causal_x, causal_edge, conf_x, conf_edge, node_score)


# speedup vs baseline: 11.6822x; 1.1308x over previous
import functools

import jax
import jax.numpy as jnp
from jax import lax
from jax.experimental import pallas as pl
from jax.experimental.pallas import tpu as pltpu
from jax.experimental.pallas import tpu_sc as plsc

_N = 10000
_E = 160000
_D = 256
_K = 5000
_TAU = 0.1
_NP = 10240
_EPAD = 163840
_BLK = 1000


def _mm_split_body(relu_in, x_ref, w_ref, b_ref, o0, o1, o2, o3):
    xb = x_ref[...]
    if relu_in:
        xb = jnp.maximum(xb, 0.0)
    g = jnp.dot(xb, w_ref[...], preferred_element_type=jnp.float32) + b_ref[...]
    o0[...] = g[:, 0:64]
    o1[...] = g[:, 64:128]
    o2[...] = g[:, 128:192]
    o3[...] = g[:, 192:256]


def _mm_split(x, w, b, relu_in):
    return pl.pallas_call(
        functools.partial(_mm_split_body, relu_in),
        grid=(_N // _BLK,),
        in_specs=[
            pl.BlockSpec((_BLK, _D), lambda g: (g, 0)),
            pl.BlockSpec((_D, _D), lambda g: (0, 0)),
            pl.BlockSpec((1, _D), lambda g: (0, 0)),
        ],
        out_specs=[pl.BlockSpec((_BLK, 64), lambda g: (g, 0))] * 4,
        out_shape=[jax.ShapeDtypeStruct((_N, 64), jnp.float32)] * 4,
    )(x, w, b.reshape(1, _D))


def _mm_splitq_body(x_ref0, x_ref1, x_ref2, x_ref3, w_ref, b_ref, o0, o1, o2, o3):
    xb = jnp.maximum(jnp.concatenate(
        [x_ref0[...], x_ref1[...], x_ref2[...], x_ref3[...]], axis=1), 0.0)
    g = jnp.dot(xb, w_ref[...], preferred_element_type=jnp.float32) + b_ref[...]
    o0[...] = g[:, 0:64]
    o1[...] = g[:, 64:128]
    o2[...] = g[:, 128:192]
    o3[...] = g[:, 192:256]


def _mm_splitq(q0, q1, q2, q3, w, b):
    return pl.pallas_call(
        _mm_splitq_body,
        grid=(_N // _BLK,),
        in_specs=[pl.BlockSpec((_BLK, 64), lambda g: (g, 0))] * 4 + [
            pl.BlockSpec((_D, _D), lambda g: (0, 0)),
            pl.BlockSpec((1, _D), lambda g: (0, 0)),
        ],
        out_specs=[pl.BlockSpec((_BLK, 64), lambda g: (g, 0))] * 4,
        out_shape=[jax.ShapeDtypeStruct((_N, 64), jnp.float32)] * 4,
    )(q0, q1, q2, q3, w, b.reshape(1, _D))


def _mm3_body(x_ref0, x_ref1, x_ref2, x_ref3, w_ref, b_ref, h_ref, s_ref):
    h = jnp.maximum(jnp.concatenate(
        [x_ref0[...], x_ref1[...], x_ref2[...], x_ref3[...]], axis=1), 0.0)
    h_ref[...] = h
    s_ref[...] = jnp.dot(h, w_ref[...], preferred_element_type=jnp.float32) + b_ref[...]


def _mm3(q0, q1, q2, q3, w3, b3):
    return pl.pallas_call(
        _mm3_body,
        grid=(_N // _BLK,),
        in_specs=[pl.BlockSpec((_BLK, 64), lambda g: (g, 0))] * 4 + [
            pl.BlockSpec((_D, 1), lambda g: (0, 0)),
            pl.BlockSpec((1, 1), lambda g: (0, 0)),
        ],
        out_specs=[
            pl.BlockSpec((_BLK, _D), lambda g: (g, 0)),
            pl.BlockSpec((_BLK, 1), lambda g: (g, 0)),
        ],
        out_shape=[
            jax.ShapeDtypeStruct((_N, _D), jnp.float32),
            jax.ShapeDtypeStruct((_N, 1), jnp.float32),
        ],
    )(q0, q1, q2, q3, w3, b3.reshape(1, 1))


def _topk_body(p_ref, ns_ref, mask_ref):
    p = p_ref[0] + p_ref[1]
    ns_ref[...] = p
    bits = lax.bitcast_convert_type(p, jnp.uint32)
    neg = (bits >> jnp.uint32(31)) == jnp.uint32(1)
    key = jnp.where(neg, ~bits, bits | jnp.uint32(0x80000000))
    row = lax.broadcasted_iota(jnp.int32, (80, 128), 0)
    col = lax.broadcasted_iota(jnp.int32, (80, 128), 1)
    idx = row * 128 + col
    real = idx < _N
    key = jnp.where(real, key, jnp.uint32(0))
    k = jnp.int32(_K)
    lo = jnp.uint32(0)
    for b in range(31, -1, -1):
        cand = lo | jnp.uint32(1 << b)
        cnt = jnp.sum((key >= cand).astype(jnp.int32))
        lo = jnp.where(cnt >= k, cand, lo)
    gt = key > lo
    eq = (key == lo) & real
    need = k - jnp.sum(gt.astype(jnp.int32))
    jv = jnp.int32(0)
    for b in range(14, -1, -1):
        cand = jv | jnp.int32(1 << b)
        g = jnp.sum((eq & (idx < cand)).astype(jnp.int32))
        jv = jnp.where(g <= need, cand, jv)
    mask = gt | (eq & (idx < jv))
    mask_ref[...] = mask.astype(jnp.float32)


def _topk(parts):
    return pl.pallas_call(
        _topk_body,
        out_shape=[jax.ShapeDtypeStruct((80, 128), jnp.float32)] * 2,
    )(parts.reshape(2, 80, 128))


def _apply_body(h_ref, c_ref, k_ref, co_ref, ko_ref):
    h = h_ref[...]
    co_ref[...] = h * c_ref[...]
    ko_ref[...] = h * k_ref[...]


def _apply(h2, cw, kw):
    return pl.pallas_call(
        _apply_body,
        grid=(_N // _BLK,),
        in_specs=[
            pl.BlockSpec((_BLK, _D), lambda g: (g, 0)),
            pl.BlockSpec((_BLK, 1), lambda g: (g, 0)),
            pl.BlockSpec((_BLK, 1), lambda g: (g, 0)),
        ],
        out_specs=[
            pl.BlockSpec((_BLK, _D), lambda g: (g, 0)),
            pl.BlockSpec((_BLK, _D), lambda g: (g, 0)),
        ],
        out_shape=[jax.ShapeDtypeStruct((_N, _D), jnp.float32)] * 2,
    )(h2, cw, kw)


@functools.cache
def _mesh():
    return plsc.VectorSubcoreMesh(core_axis_name="c", subcore_axis_name="s")


@functools.cache
def _rowseg_call():
    return functools.partial(
        pl.kernel,
        mesh=_mesh(),
        out_type=[jax.ShapeDtypeStruct((_NP, 64), jnp.float32)] * 4,
        scratch_types=[
            pltpu.VMEM((80, 125), jnp.int32),
            pltpu.VMEM((80, 125), jnp.int32),
            pltpu.VMEM((125, 64), jnp.float32),
            pltpu.VMEM((125, 64), jnp.float32),
            pltpu.VMEM((125, 64), jnp.float32),
            pltpu.VMEM((125, 64), jnp.float32),
            pltpu.VMEM((128, 64), jnp.float32),
            pltpu.VMEM_SHARED((_NP, 64), jnp.float32),
            pltpu.SemaphoreType.DMA,
            pltpu.SemaphoreType.DMA,
            pltpu.SemaphoreType.DMA,
            pltpu.SemaphoreType.DMA,
        ],
        compiler_params=pltpu.CompilerParams(use_tc_tiling_on_sc=False),
    )(_rowseg)


def _rowseg(g0, g1, g2, g3, srcT, dstT, zz, o0, o1, o2, o3, srcbuf, dstbuf,
            r0, r1, r2, r3, obuf, acc, s0, s1, s2, s3):
    c = lax.axis_index("c")
    t = lax.axis_index("s")
    pltpu.sync_copy(srcT.at[t], srcbuf)
    pltpu.sync_copy(dstT.at[t], dstbuf)

    def _round(tbl, oref):
        pltpu.sync_copy(zz, obuf)
        for q in range(5):
            pltpu.sync_copy(obuf, acc.at[pl.ds(t * 640 + q * 128, 128), :])
        plsc.subcore_barrier()

        bufs = ((r0, s0), (r1, s1), (r2, s2), (r3, s3))
        for u in range(3):
            pltpu.async_copy(tbl.at[srcbuf.at[u]], bufs[u][0], bufs[u][1])

        def body(jj, carry):
            j = jj * 4
            for u in range(4):
                bufp, smp = bufs[(u + 3) % 4]
                pltpu.async_copy(tbl.at[srcbuf.at[lax.rem(j + u + 3, 80)]],
                                 bufp, smp)
                buf, sm = bufs[u]
                pltpu.make_async_copy(tbl.at[srcbuf.at[0]], buf, sm).wait()
                pltpu.sync_copy(buf, acc.at[dstbuf.at[j + u]], add=True)
            return carry
        lax.fori_loop(0, 20, body, 0)
        for u in range(3):
            pltpu.make_async_copy(tbl.at[srcbuf.at[0]], bufs[u][0],
                                  bufs[u][1]).wait()
        plsc.subcore_barrier()

        pltpu.sync_copy(acc.at[pl.ds(t * 640, 640), :],
                        oref.at[pl.ds(t * 640, 640), :])
        plsc.subcore_barrier()

    @pl.when(c == 0)
    def _():
        _round(g0, o0)
        _round(g1, o1)

    @pl.when(c == 1)
    def _():
        _round(g2, o2)
        _round(g3, o3)


@functools.cache
def _scalseg_call():
    return functools.partial(
        pl.kernel,
        mesh=_mesh(),
        out_type=jax.ShapeDtypeStruct((2, _NP), jnp.float32),
        scratch_types=[
            pltpu.VMEM((40, 128), jnp.int32),
            pltpu.VMEM((40, 128), jnp.int32),
            pltpu.VMEM((128,), jnp.float32),
            pltpu.VMEM((128,), jnp.float32),
            pltpu.VMEM((640,), jnp.float32),
            pltpu.VMEM_SHARED((_NP,), jnp.float32),
            pltpu.SemaphoreType.DMA,
            pltpu.SemaphoreType.DMA,
        ],
    )(_scalseg)


def _scalseg(s_ext, srcE, dstE, out, srcbuf, dstbuf, valrow, valrow2, zb, acc,
             sem, sem2):
    c = lax.axis_index("c")
    t = lax.axis_index("s")
    w = c * 16 + t

    def zbody(i, carry):
        zb[pl.ds(i * 16, 16)] = jnp.zeros((16,), jnp.float32)
        return carry
    lax.fori_loop(0, 40, zbody, 0)
    pltpu.sync_copy(zb, acc.at[pl.ds(t * 640, 640)])
    pltpu.sync_copy(srcE.at[w], srcbuf)
    pltpu.sync_copy(dstE.at[w], dstbuf)
    plsc.subcore_barrier()

    pltpu.async_copy(s_ext.at[srcbuf.at[0]], valrow, sem)

    def body(jj, carry):
        j = jj * 2
        pltpu.async_copy(s_ext.at[srcbuf.at[j + 1]], valrow2, sem2)
        pltpu.make_async_copy(s_ext.at[srcbuf.at[0]], valrow, sem).wait()
        pltpu.sync_copy(valrow, acc.at[dstbuf.at[j]], add=True)
        pltpu.async_copy(s_ext.at[srcbuf.at[lax.rem(j + 2, 40)]], valrow, sem)
        pltpu.make_async_copy(s_ext.at[srcbuf.at[0]], valrow2, sem2).wait()
        pltpu.sync_copy(valrow2, acc.at[dstbuf.at[j + 1]], add=True)
        return carry
    lax.fori_loop(0, 20, body, 0)
    pltpu.make_async_copy(s_ext.at[srcbuf.at[0]], valrow, sem).wait()

    plsc.subcore_barrier()
    pltpu.sync_copy(acc.at[pl.ds(t * 640, 640)], zb)
    pltpu.sync_copy(zb, out.at[c, pl.ds(t * 640, 640)])


@functools.cache
def _edgemask_call():
    return functools.partial(
        pl.kernel,
        mesh=_mesh(),
        out_type=[jax.ShapeDtypeStruct((32, 40, 128), jnp.float32)] * 2,
        scratch_types=[
            pltpu.VMEM((40, 128), jnp.int32),
            pltpu.VMEM((40, 128), jnp.int32),
            pltpu.VMEM((128,), jnp.float32),
            pltpu.VMEM((128,), jnp.float32),
            pltpu.VMEM((128,), jnp.float32),
            pltpu.VMEM((128,), jnp.float32),
            pltpu.VMEM((40, 128), jnp.float32),
            pltpu.VMEM((40, 128), jnp.float32),
            pltpu.SemaphoreType.DMA,
            pltpu.SemaphoreType.DMA,
            pltpu.SemaphoreType.DMA,
            pltpu.SemaphoreType.DMA,
        ],
    )(_edgemask)


def _edgemask(mext, srcE, dstE, ce_out, ke_out, srcbuf, dstbuf, msrow, mdrow,
              msrow2, mdrow2, cbuf, kbuf, sem, sem2, sem3, sem4):
    c = lax.axis_index("c")
    t = lax.axis_index("s")
    w = c * 16 + t
    pltpu.sync_copy(srcE.at[w], srcbuf)
    pltpu.sync_copy(dstE.at[w], dstbuf)
    one = jnp.ones((16,), jnp.float32)

    def compute(j, ms_ref, md_ref):
        for u in range(8):
            sl = pl.ds(u * 16, 16)
            ms = ms_ref[sl]
            md = md_ref[sl]
            cbuf[j, sl] = ms * md
            kbuf[j, sl] = (one - ms) * (one - md)

    pltpu.async_copy(mext.at[srcbuf.at[0]], msrow, sem)
    pltpu.async_copy(mext.at[dstbuf.at[0]], mdrow, sem2)

    def body(jj, carry):
        j = jj * 2
        pltpu.async_copy(mext.at[srcbuf.at[j + 1]], msrow2, sem3)
        pltpu.async_copy(mext.at[dstbuf.at[j + 1]], mdrow2, sem4)
        pltpu.make_async_copy(mext.at[srcbuf.at[0]], msrow, sem).wait()
        pltpu.make_async_copy(mext.at[dstbuf.at[0]], mdrow, sem2).wait()
        compute(j, msrow, mdrow)
        jn = lax.rem(j + 2, 40)
        pltpu.async_copy(mext.at[srcbuf.at[jn]], msrow, sem)
        pltpu.async_copy(mext.at[dstbuf.at[jn]], mdrow, sem2)
        pltpu.make_async_copy(mext.at[srcbuf.at[0]], msrow2, sem3).wait()
        pltpu.make_async_copy(mext.at[dstbuf.at[0]], mdrow2, sem4).wait()
        compute(j + 1, msrow2, mdrow2)
        return carry
    lax.fori_loop(0, 20, body, 0)
    pltpu.make_async_copy(mext.at[srcbuf.at[0]], msrow, sem).wait()
    pltpu.make_async_copy(mext.at[dstbuf.at[0]], mdrow, sem2).wait()
    pltpu.sync_copy(cbuf, ce_out.at[w])
    pltpu.sync_copy(kbuf, ke_out.at[w])


def kernel(x, edge, W1, b1, W2, b2, W3, b3):
    src = edge[0].astype(jnp.int32)
    dst = edge[1].astype(jnp.int32)
    srcT = src.reshape(16, 80, 125)
    dstT = dst.reshape(16, 80, 125)
    pad = jnp.arange(_EPAD - _E, dtype=jnp.int32) % (_NP - _N) + _N
    srcE = jnp.concatenate([src, pad]).reshape(32, 40, 128)
    dstE = jnp.concatenate([dst, pad]).reshape(32, 40, 128)
    zz = jnp.zeros((128, 64), jnp.float32)

    g1q = _mm_split(x, W1, b1, relu_in=False)
    A1q = _rowseg_call()(*g1q, srcT, dstT, zz)
    g2q = _mm_splitq(*A1q, W2, b2)
    A2q = _rowseg_call()(*g2q, srcT, dstT, zz)
    h2, s = _mm3(*A2q, W3, b3)

    s_ext = jnp.concatenate([s[:, 0], jnp.zeros((_NP - _N,), jnp.float32)])
    parts = _scalseg_call()(s_ext, srcE, dstE)
    ns80, mask80 = _topk(parts)
    ns = ns80.reshape(_NP)[:_N]
    node_score = ns[:, None]
    maskv = mask80.reshape(_NP)[:_N]

    w = jax.nn.sigmoid(ns / _TAU)
    cw = (maskv * w)[:, None]
    kw = ((1.0 - maskv) * (1.0 - w))[:, None]
    causal_x, conf_x = _apply(h2, cw, kw)

    mext = jnp.concatenate([maskv, jnp.zeros((_NP - _N,), jnp.float32)])
    ce, ke = _edgemask_call()(mext, srcE, dstE)
    causal_edge = ce.reshape(_EPAD)[:_E]
    conf_edge = ke.reshape(_EPAD)[:_E]
    return (causal_x, causal_edge, conf_x, conf_edge, node_score)
